# baseline (speedup 1.0000x reference)
import functools

import jax
import jax.numpy as jnp
from jax import lax
from jax.experimental import pallas as pl
from jax.experimental.pallas import tpu as pltpu
from jax.experimental.pallas import tpu_sc as plsc

N_OUT = 65536
LANES = 16
NC, NS = 2, 16
NW = NC * NS
PER_W = N_OUT // NW
VECS = PER_W // LANES


def _seen_body(csr_hbm, out_hbm, buf_v, out_v):
    wid = lax.axis_index("s") * NC + lax.axis_index("c")
    base = wid * PER_W
    pltpu.sync_copy(csr_hbm.at[pl.ds(base, PER_W + 1)], buf_v)

    def body(j, carry):
        lo = buf_v[pl.ds(j * LANES, LANES)]
        hi = buf_v[pl.ds(j * LANES + 1, LANES)]
        out_v[pl.ds(j * LANES, LANES)] = jnp.where(
            hi > lo,
            jnp.full((LANES,), 1, jnp.int32),
            jnp.zeros((LANES,), jnp.int32),
        )
        return carry

    lax.fori_loop(0, VECS, body, 0)
    pltpu.sync_copy(out_v, out_hbm.at[pl.ds(base, PER_W)])


_seen = functools.partial(
    pl.kernel,
    out_type=jax.ShapeDtypeStruct((N_OUT,), jnp.int32),
    mesh=plsc.VectorSubcoreMesh(core_axis_name="c", subcore_axis_name="s"),
    scratch_types=[
        pltpu.VMEM((PER_W + 1,), jnp.int32),
        pltpu.VMEM((PER_W,), jnp.int32),
    ],
)(_seen_body)


def _copy_body(x3d_ref, modx_ref, o1_ref, o2_ref, sem1, sem2):
    c1 = pltpu.make_async_copy(x3d_ref, o1_ref, sem1)
    c2 = pltpu.make_async_copy(modx_ref, o2_ref, sem2)
    c1.start()
    c2.start()
    c1.wait()
    c2.wait()


def _dense_copy(x_3d, mod_x):
    return pl.pallas_call(
        _copy_body,
        in_specs=[
            pl.BlockSpec(memory_space=pl.ANY),
            pl.BlockSpec(memory_space=pl.ANY),
        ],
        out_specs=[
            pl.BlockSpec(memory_space=pl.ANY),
            pl.BlockSpec(memory_space=pl.ANY),
        ],
        out_shape=[
            jax.ShapeDtypeStruct(x_3d.shape, x_3d.dtype),
            jax.ShapeDtypeStruct(mod_x.shape, mod_x.dtype),
        ],
        scratch_shapes=[pltpu.SemaphoreType.DMA, pltpu.SemaphoreType.DMA],
    )(x_3d, mod_x)


def kernel(x_3d, mod_x, csr_idx):
    seen = _seen(csr_idx.astype(jnp.int32))
    x_3d_out, mod_x_out = _dense_copy(x_3d, mod_x)
    return (x_3d_out, mod_x_out, seen.astype(jnp.bool_))


# speedup vs baseline: 35.8887x; 35.8887x over previous
import functools

import jax
import jax.numpy as jnp
from jax import lax
from jax.experimental import pallas as pl
from jax.experimental.pallas import tpu as pltpu
from jax.experimental.pallas import tpu_sc as plsc

N_OUT = 65536
LANES = 16
NC, NS = 2, 16
NW = NC * NS
PER_W = N_OUT // NW
VECS = PER_W // LANES


def _seen_body(csr_hbm, out_hbm, buf_v, out_v):
    wid = lax.axis_index("s") * NC + lax.axis_index("c")
    base = wid * PER_W
    pltpu.sync_copy(csr_hbm.at[pl.ds(base, PER_W + 1)], buf_v)

    def body(j, carry):
        lo = buf_v[pl.ds(j * LANES, LANES)]
        hi = buf_v[pl.ds(j * LANES + 1, LANES)]
        out_v[pl.ds(j * LANES, LANES)] = jnp.where(
            hi > lo,
            jnp.full((LANES,), 1, jnp.int32),
            jnp.zeros((LANES,), jnp.int32),
        )
        return carry

    lax.fori_loop(0, VECS, body, 0)
    pltpu.sync_copy(out_v, out_hbm.at[pl.ds(base, PER_W)])


_seen = functools.partial(
    pl.kernel,
    out_type=jax.ShapeDtypeStruct((N_OUT,), jnp.int32),
    mesh=plsc.VectorSubcoreMesh(core_axis_name="c", subcore_axis_name="s"),
    scratch_types=[
        pltpu.VMEM((PER_W + 1,), jnp.int32),
        pltpu.VMEM((PER_W,), jnp.int32),
    ],
)(_seen_body)


_G = 128


def _copy_body(x3d_ref, modx_ref, o1_ref, o2_ref):
    o1_ref[...] = x3d_ref[...]
    o2_ref[...] = modx_ref[...]


def _dense_copy(x_3d, mod_x):
    r1 = x_3d.shape[0] // _G
    r2 = mod_x.shape[0] // _G
    return pl.pallas_call(
        _copy_body,
        grid=(_G,),
        in_specs=[
            pl.BlockSpec((r1, x_3d.shape[1]), lambda g: (g, 0)),
            pl.BlockSpec((r2, mod_x.shape[1]), lambda g: (g, 0)),
        ],
        out_specs=[
            pl.BlockSpec((r1, x_3d.shape[1]), lambda g: (g, 0)),
            pl.BlockSpec((r2, mod_x.shape[1]), lambda g: (g, 0)),
        ],
        out_shape=[
            jax.ShapeDtypeStruct(x_3d.shape, x_3d.dtype),
            jax.ShapeDtypeStruct(mod_x.shape, mod_x.dtype),
        ],
    )(x_3d, mod_x)


def kernel(x_3d, mod_x, csr_idx):
    seen = _seen(csr_idx.astype(jnp.int32))
    x_3d_out, mod_x_out = _dense_copy(x_3d, mod_x)
    return (x_3d_out, mod_x_out, seen.astype(jnp.bool_))


# speedup vs baseline: 42.7111x; 1.1901x over previous
import functools

import jax
import jax.numpy as jnp
from jax import lax
from jax.experimental import pallas as pl
from jax.experimental.pallas import tpu as pltpu
from jax.experimental.pallas import tpu_sc as plsc

N_OUT = 65536
LANES = 16
NC, NS = 2, 16
NW = NC * NS
PER_W = N_OUT // NW
VECS = PER_W // LANES


def _seen_body(csr_hbm, out_hbm, buf_v, out_v):
    wid = lax.axis_index("s") * NC + lax.axis_index("c")
    base = wid * PER_W
    pltpu.sync_copy(csr_hbm.at[pl.ds(base, PER_W + 1)], buf_v)

    def body(j, carry):
        lo = buf_v[pl.ds(j * LANES, LANES)]
        hi = buf_v[pl.ds(j * LANES + 1, LANES)]
        out_v[pl.ds(j * LANES, LANES)] = jnp.where(
            hi > lo,
            jnp.full((LANES,), 1, jnp.int32),
            jnp.zeros((LANES,), jnp.int32),
        )
        return carry

    lax.fori_loop(0, VECS, body, 0)
    pltpu.sync_copy(out_v, out_hbm.at[pl.ds(base, PER_W)])


_seen = functools.partial(
    pl.kernel,
    out_type=jax.ShapeDtypeStruct((N_OUT,), jnp.int32),
    mesh=plsc.VectorSubcoreMesh(core_axis_name="c", subcore_axis_name="s"),
    scratch_types=[
        pltpu.VMEM((PER_W + 1,), jnp.int32),
        pltpu.VMEM((PER_W,), jnp.int32),
    ],
)(_seen_body)


_G = 32


def _copy_body(x3d_ref, modx_ref, o1_ref, o2_ref):
    o1_ref[...] = x3d_ref[...]
    o2_ref[...] = modx_ref[...]


def _dense_copy(x_3d, mod_x):
    r1 = x_3d.shape[0] // _G
    r2 = mod_x.shape[0] // _G
    return pl.pallas_call(
        _copy_body,
        grid=(_G,),
        in_specs=[
            pl.BlockSpec((r1, x_3d.shape[1]), lambda g: (g, 0)),
            pl.BlockSpec((r2, mod_x.shape[1]), lambda g: (g, 0)),
        ],
        out_specs=[
            pl.BlockSpec((r1, x_3d.shape[1]), lambda g: (g, 0)),
            pl.BlockSpec((r2, mod_x.shape[1]), lambda g: (g, 0)),
        ],
        out_shape=[
            jax.ShapeDtypeStruct(x_3d.shape, x_3d.dtype),
            jax.ShapeDtypeStruct(mod_x.shape, mod_x.dtype),
        ],
    )(x_3d, mod_x)


def kernel(x_3d, mod_x, csr_idx):
    seen = _seen(csr_idx.astype(jnp.int32))
    x_3d_out, mod_x_out = _dense_copy(x_3d, mod_x)
    return (x_3d_out, mod_x_out, seen.astype(jnp.bool_))
